# baseline (speedup 1.0000x reference)
import functools

import jax
import jax.numpy as jnp
from jax import lax
from jax.experimental import pallas as pl
from jax.experimental.pallas import tpu as pltpu
from jax.experimental.pallas import tpu_sc as plsc

E = 64
K = 8
L = 16


def _logits_body(x_ref, w_ref, out_ref):
    out_ref[...] = jnp.dot(x_ref[...], w_ref[...],
                           preferred_element_type=jnp.float32)


def _router_logits(x2d, w):
    t, h = x2d.shape
    bt = 512
    return pl.pallas_call(
        _logits_body,
        grid=(t // bt,),
        in_specs=[
            pl.BlockSpec((bt, h), lambda i: (i, 0)),
            pl.BlockSpec((h, E), lambda i: (0, 0)),
        ],
        out_specs=pl.BlockSpec((bt, E), lambda i: (i, 0)),
        out_shape=jax.ShapeDtypeStruct((t, E), jnp.float32),
    )(x2d, w)


def _make_topk_sc(t, interpret=False):
    nc, ns = 2, 16
    nw = nc * ns
    tpw = t // nw
    ncol = tpw // L

    mesh = plsc.VectorSubcoreMesh(core_axis_name="c", subcore_axis_name="s",
                                  num_cores=nc, num_subcores=ns)

    @functools.partial(
        pl.kernel,
        out_type=(
            jax.ShapeDtypeStruct((t, K), jnp.float32),
            jax.ShapeDtypeStruct((t, K), jnp.int32),
        ),
        mesh=mesh,
        scratch_types=[
            pltpu.VMEM((tpw, E), jnp.float32),
            pltpu.VMEM((tpw, K), jnp.float32),
            pltpu.VMEM((tpw, K), jnp.int32),
        ],
        compiler_params=pltpu.CompilerParams(needs_layout_passes=False),
        interpret=interpret,
    )
    def topk(logits_hbm, w_hbm, i_hbm, loc, wloc, iloc):
        wid = lax.axis_index("s") * nc + lax.axis_index("c")
        base = wid * tpw
        pltpu.sync_copy(logits_hbm.at[pl.ds(base, tpw), :], loc)

        lane = lax.iota(jnp.int32, L)
        neg_inf = jnp.full((L,), -jnp.inf, jnp.float32)
        zero_i = jnp.zeros((L,), jnp.int32)

        for c in range(ncol):
            tok = c * L + lane

            def body(e, carry):
                vs = list(carry[0])
                ix = list(carry[1])
                eidx = jnp.full((L,), e, jnp.int32)
                val = plsc.load_gather(loc, [tok, eidx])
                cur_v, cur_i = val, eidx
                for kk in range(K):
                    m = cur_v > vs[kk]
                    nv = jnp.where(m, cur_v, vs[kk])
                    ni = jnp.where(m, cur_i, ix[kk])
                    cur_v = jnp.where(m, vs[kk], cur_v)
                    cur_i = jnp.where(m, ix[kk], cur_i)
                    vs[kk] = nv
                    ix[kk] = ni
                return (tuple(vs), tuple(ix))

            init = (tuple(neg_inf for _ in range(K)),
                    tuple(zero_i for _ in range(K)))
            vs, ix = lax.fori_loop(0, E, body, init)

            mx = vs[0]
            es = [jnp.exp(v - mx) for v in vs]
            tot = es[0]
            for kk in range(1, K):
                tot = tot + es[kk]
            inv = 1.0 / tot
            for kk in range(K):
                kidx = jnp.full((L,), kk, jnp.int32)
                plsc.store_scatter(wloc, [tok, kidx], es[kk] * inv)
                plsc.store_scatter(iloc, [tok, kidx], ix[kk])

        pltpu.sync_copy(wloc, w_hbm.at[pl.ds(base, tpw), :])
        pltpu.sync_copy(iloc, i_hbm.at[pl.ds(base, tpw), :])

    return topk


def kernel(x, W_router):
    b, s, h = x.shape
    t = b * s
    x2d = x.reshape(t, h)
    logits = _router_logits(x2d, W_router)
    weights, indices = _make_topk_sc(t)(logits)
    return (weights.reshape(b, s, K),
            indices.reshape(b, s, K),
            logits.reshape(b, s, E))
